# baseline (speedup 1.0000x reference)
import jax
import jax.numpy as jnp
from jax import lax
from jax.experimental import pallas as pl
from jax.experimental.pallas import tpu as pltpu

_N = 4096
_D = 32
_KP1 = 51
_BLK = 128
_GRID = _N // _BLK


def _body(f_ref, w0_ref, b0_ref, w1_ref, b1_ref, o_ref, emb_ref, s_ref, k_ref,
          kh_ref):
    i = pl.program_id(0)

    @pl.when(i == 0)
    def _():
        f = f_ref[...]
        h = lax.dot_general(f, w0_ref[...], (((1,), (1,)), ((), ())),
                            preferred_element_type=jnp.float32) + b0_ref[...]
        h = jnp.maximum(h, 0.0)
        h = lax.dot_general(h, w1_ref[...], (((1,), (1,)), ((), ())),
                            preferred_element_type=jnp.float32) + b1_ref[...]
        nrm = jnp.sqrt(jnp.sum(h * h, axis=1, keepdims=True))
        emb_ref[...] = h / jnp.maximum(nrm, 1e-12)

    rows = emb_ref[pl.ds(i * _BLK, _BLK), :]
    s = lax.dot_general(rows, emb_ref[...], (((1,), (1,)), ((), ())),
                        preferred_element_type=jnp.float32)
    r = jnp.maximum(s, 0.0)
    s_ref[...] = r

    k21 = jnp.minimum(lax.convert_element_type(r * 2097152.0, jnp.int32),
                      jnp.int32(2097151))
    k_ref[...] = k21
    kh_ref[...] = lax.convert_element_type(k21 >> 6, jnp.int16)

    def count_ge16(cand):
        cand16 = lax.convert_element_type(cand, jnp.int16)
        m = jnp.where(kh_ref[...] >= cand16, jnp.int16(1), jnp.int16(0))
        acc = m[:, 0:256]
        for c in range(1, 16):
            acc = acc + m[:, c * 256:(c + 1) * 256]
        return jnp.sum(acc.astype(jnp.int32), axis=1, keepdims=True)

    ph = jnp.zeros((_BLK, 1), jnp.int32)
    for bit in range(14, -1, -1):
        cand = ph | jnp.int32(1 << bit)
        ph = jnp.where(count_ge16(cand) >= _KP1, cand, ph)
    base = ph << 6

    kh_ref[...] = lax.convert_element_type(
        jnp.clip(k_ref[...] - base, -1, 64), jnp.int16)

    pb = jnp.zeros((_BLK, 1), jnp.int32)
    for bit in range(5, -1, -1):
        cand = pb | jnp.int32(1 << bit)
        pb = jnp.where(count_ge16(cand) >= _KP1, cand, pb)

    thr = lax.convert_element_type(base + pb, jnp.float32) * (2.0 ** -21)
    rr = s_ref[...]
    o_ref[...] = jnp.where(rr >= thr, rr, 0.0)


def kernel(features, W0, b0, W1, b1):
    b0r = b0.reshape(1, _D)
    b1r = b1.reshape(1, _D)
    return pl.pallas_call(
        _body,
        grid=(_GRID,),
        in_specs=[
            pl.BlockSpec((_N, _D), lambda i: (0, 0)),
            pl.BlockSpec((_D, _D), lambda i: (0, 0)),
            pl.BlockSpec((1, _D), lambda i: (0, 0)),
            pl.BlockSpec((_D, _D), lambda i: (0, 0)),
            pl.BlockSpec((1, _D), lambda i: (0, 0)),
        ],
        out_specs=pl.BlockSpec((_BLK, _N), lambda i: (i, 0)),
        out_shape=jax.ShapeDtypeStruct((_N, _N), jnp.float32),
        scratch_shapes=[
            pltpu.VMEM((_N, _D), jnp.float32),
            pltpu.VMEM((_BLK, _N), jnp.float32),
            pltpu.VMEM((_BLK, _N), jnp.int32),
            pltpu.VMEM((_BLK, _N), jnp.int16),
        ],
    )(features, W0, b0r, W1, b1r)


# speedup vs baseline: 1.0408x; 1.0408x over previous
import jax
import jax.numpy as jnp
from jax import lax
from jax.experimental import pallas as pl
from jax.experimental.pallas import tpu as pltpu

_N = 4096
_D = 32
_KP1 = 51
_BLK = 256
_GRID = _N // _BLK


def _body(f_ref, w0_ref, b0_ref, w1_ref, b1_ref, o_ref, emb_ref, s_ref, k_ref,
          kh_ref):
    i = pl.program_id(0)

    @pl.when(i == 0)
    def _():
        f = f_ref[...]
        h = lax.dot_general(f, w0_ref[...], (((1,), (1,)), ((), ())),
                            preferred_element_type=jnp.float32) + b0_ref[...]
        h = jnp.maximum(h, 0.0)
        h = lax.dot_general(h, w1_ref[...], (((1,), (1,)), ((), ())),
                            preferred_element_type=jnp.float32) + b1_ref[...]
        nrm = jnp.sqrt(jnp.sum(h * h, axis=1, keepdims=True))
        emb_ref[...] = h / jnp.maximum(nrm, 1e-12)

    rows = emb_ref[pl.ds(i * _BLK, _BLK), :]
    s = lax.dot_general(rows, emb_ref[...], (((1,), (1,)), ((), ())),
                        preferred_element_type=jnp.float32)
    r = jnp.maximum(s, 0.0)
    s_ref[...] = r

    k21 = jnp.minimum(lax.convert_element_type(r * 2097152.0, jnp.int32),
                      jnp.int32(2097151))
    k_ref[...] = k21
    kh_ref[...] = lax.convert_element_type(k21 >> 6, jnp.int16)

    def count_ge16(cand):
        cand16 = lax.convert_element_type(cand, jnp.int16)
        m = jnp.where(kh_ref[...] >= cand16, jnp.int16(1), jnp.int16(0))
        acc = m[:, 0:256]
        for c in range(1, 16):
            acc = acc + m[:, c * 256:(c + 1) * 256]
        return jnp.sum(acc.astype(jnp.int32), axis=1, keepdims=True)

    ph = jnp.zeros((_BLK, 1), jnp.int32)
    for bit in range(14, -1, -1):
        cand = ph | jnp.int32(1 << bit)
        ph = jnp.where(count_ge16(cand) >= _KP1, cand, ph)
    base = ph << 6

    kh_ref[...] = lax.convert_element_type(
        jnp.clip(k_ref[...] - base, -1, 64), jnp.int16)

    pb = jnp.zeros((_BLK, 1), jnp.int32)
    for bit in range(5, -1, -1):
        cand = pb | jnp.int32(1 << bit)
        pb = jnp.where(count_ge16(cand) >= _KP1, cand, pb)

    thr = lax.convert_element_type(base + pb, jnp.float32) * (2.0 ** -21)
    rr = s_ref[...]
    o_ref[...] = jnp.where(rr >= thr, rr, 0.0)


def kernel(features, W0, b0, W1, b1):
    b0r = b0.reshape(1, _D)
    b1r = b1.reshape(1, _D)
    return pl.pallas_call(
        _body,
        grid=(_GRID,),
        in_specs=[
            pl.BlockSpec((_N, _D), lambda i: (0, 0)),
            pl.BlockSpec((_D, _D), lambda i: (0, 0)),
            pl.BlockSpec((1, _D), lambda i: (0, 0)),
            pl.BlockSpec((_D, _D), lambda i: (0, 0)),
            pl.BlockSpec((1, _D), lambda i: (0, 0)),
        ],
        out_specs=pl.BlockSpec((_BLK, _N), lambda i: (i, 0)),
        out_shape=jax.ShapeDtypeStruct((_N, _N), jnp.float32),
        scratch_shapes=[
            pltpu.VMEM((_N, _D), jnp.float32),
            pltpu.VMEM((_BLK, _N), jnp.float32),
            pltpu.VMEM((_BLK, _N), jnp.int32),
            pltpu.VMEM((_BLK, _N), jnp.int16),
        ],
    )(features, W0, b0r, W1, b1r)


# speedup vs baseline: 1.0593x; 1.0178x over previous
import jax
import jax.numpy as jnp
from jax import lax
from jax.experimental import pallas as pl
from jax.experimental.pallas import tpu as pltpu

_N = 4096
_D = 32
_KP1 = 51
_BLK = 512
_GRID = _N // _BLK


def _body(f_ref, w0_ref, b0_ref, w1_ref, b1_ref, o_ref, emb_ref, s_ref, k_ref,
          kh_ref):
    i = pl.program_id(0)

    @pl.when(i == 0)
    def _():
        f = f_ref[...]
        h = lax.dot_general(f, w0_ref[...], (((1,), (1,)), ((), ())),
                            preferred_element_type=jnp.float32) + b0_ref[...]
        h = jnp.maximum(h, 0.0)
        h = lax.dot_general(h, w1_ref[...], (((1,), (1,)), ((), ())),
                            preferred_element_type=jnp.float32) + b1_ref[...]
        nrm = jnp.sqrt(jnp.sum(h * h, axis=1, keepdims=True))
        emb_ref[...] = h / jnp.maximum(nrm, 1e-12)

    rows = emb_ref[pl.ds(i * _BLK, _BLK), :]
    s = lax.dot_general(rows, emb_ref[...], (((1,), (1,)), ((), ())),
                        preferred_element_type=jnp.float32)
    r = jnp.maximum(s, 0.0)
    s_ref[...] = r

    k21 = jnp.minimum(lax.convert_element_type(r * 2097152.0, jnp.int32),
                      jnp.int32(2097151))
    k_ref[...] = k21
    kh_ref[...] = lax.convert_element_type(k21 >> 6, jnp.int16)

    def count_ge16(cand):
        cand16 = lax.convert_element_type(cand, jnp.int16)
        m = jnp.where(kh_ref[...] >= cand16, jnp.int16(1), jnp.int16(0))
        acc = m[:, 0:256]
        for c in range(1, 16):
            acc = acc + m[:, c * 256:(c + 1) * 256]
        return jnp.sum(acc.astype(jnp.int32), axis=1, keepdims=True)

    ph = jnp.zeros((_BLK, 1), jnp.int32)
    for bit in range(14, -1, -1):
        cand = ph | jnp.int32(1 << bit)
        ph = jnp.where(count_ge16(cand) >= _KP1, cand, ph)
    base = ph << 6

    kh_ref[...] = lax.convert_element_type(
        jnp.clip(k_ref[...] - base, -1, 64), jnp.int16)

    pb = jnp.zeros((_BLK, 1), jnp.int32)
    for bit in range(5, -1, -1):
        cand = pb | jnp.int32(1 << bit)
        pb = jnp.where(count_ge16(cand) >= _KP1, cand, pb)

    thr = lax.convert_element_type(base + pb, jnp.float32) * (2.0 ** -21)
    rr = s_ref[...]
    o_ref[...] = jnp.where(rr >= thr, rr, 0.0)


def kernel(features, W0, b0, W1, b1):
    b0r = b0.reshape(1, _D)
    b1r = b1.reshape(1, _D)
    return pl.pallas_call(
        _body,
        grid=(_GRID,),
        in_specs=[
            pl.BlockSpec((_N, _D), lambda i: (0, 0)),
            pl.BlockSpec((_D, _D), lambda i: (0, 0)),
            pl.BlockSpec((1, _D), lambda i: (0, 0)),
            pl.BlockSpec((_D, _D), lambda i: (0, 0)),
            pl.BlockSpec((1, _D), lambda i: (0, 0)),
        ],
        out_specs=pl.BlockSpec((_BLK, _N), lambda i: (i, 0)),
        out_shape=jax.ShapeDtypeStruct((_N, _N), jnp.float32),
        scratch_shapes=[
            pltpu.VMEM((_N, _D), jnp.float32),
            pltpu.VMEM((_BLK, _N), jnp.float32),
            pltpu.VMEM((_BLK, _N), jnp.int32),
            pltpu.VMEM((_BLK, _N), jnp.int16),
        ],
    )(features, W0, b0r, W1, b1r)
